# baseline (speedup 1.0000x reference)
import jax
import jax.numpy as jnp
from jax.experimental import pallas as pl
from jax.experimental.pallas import tpu as pltpu

_THRESHOLD = 25.0
_BM = 512


def _body(x_ref, W1_ref, b1_ref, WcT_ref, bc_ref, W2_hbm, b2_ref, WfT_hbm,
          bf_ref, outT_ref, w2_vmem, wfT_vmem, sem2, semf):
    i = pl.program_id(0)

    @pl.when(i == 0)
    def _start_weight_fetch():
        pltpu.make_async_copy(W2_hbm, w2_vmem, sem2).start()
        pltpu.make_async_copy(WfT_hbm, wfT_vmem, semf).start()

    x = x_ref[...]
    h = jnp.maximum(
        jnp.dot(x, W1_ref[...], preferred_element_type=jnp.float32)
        + b1_ref[...], 0.0)
    hT = h.T
    cpT = (jnp.dot(WcT_ref[...], hT, preferred_element_type=jnp.float32)
           + bc_ref[...].T)
    mxT = jnp.max(jnp.exp(cpT), axis=0, keepdims=True)

    @pl.when(i == 0)
    def _wait_weight_fetch():
        pltpu.make_async_copy(W2_hbm, w2_vmem, sem2).wait()
        pltpu.make_async_copy(WfT_hbm, wfT_vmem, semf).wait()

    h2 = jnp.maximum(
        jnp.dot(h, w2_vmem[...], preferred_element_type=jnp.float32)
        + b2_ref[...], 0.0)
    foT = (jnp.dot(wfT_vmem[...], h2.T, preferred_element_type=jnp.float32)
           + bf_ref[...].T)
    outT_ref[...] = jnp.where(mxT > _THRESHOLD, cpT, foT)


def kernel(x, W1, b1, Wc, bc, W2, b2, Wf, bf):
    B, D = x.shape
    NC = Wc.shape[1]

    def _full(shape):
        return pl.BlockSpec(shape, lambda i: (0, 0))

    outT = pl.pallas_call(
        _body,
        grid=(B // _BM,),
        in_specs=[
            pl.BlockSpec((_BM, D), lambda i: (i, 0)),
            _full((D, D)),
            _full((1, D)),
            _full((NC, D)),
            _full((1, NC)),
            pl.BlockSpec(memory_space=pl.ANY),
            _full((1, D)),
            pl.BlockSpec(memory_space=pl.ANY),
            _full((1, NC)),
        ],
        out_specs=pl.BlockSpec((NC, _BM), lambda i: (0, i)),
        out_shape=jax.ShapeDtypeStruct((NC, B), jnp.float32),
        scratch_shapes=[
            pltpu.VMEM((D, D), jnp.float32),
            pltpu.VMEM((NC, D), jnp.float32),
            pltpu.SemaphoreType.DMA,
            pltpu.SemaphoreType.DMA,
        ],
        compiler_params=pltpu.CompilerParams(
            dimension_semantics=("arbitrary",)),
    )(x, W1, b1.reshape(1, D), Wc.T, bc.reshape(1, NC), W2,
      b2.reshape(1, D), Wf.T, bf.reshape(1, NC))
    return outT.T


# speedup vs baseline: 1.0473x; 1.0473x over previous
import jax
import jax.numpy as jnp
from jax.experimental import pallas as pl
from jax.experimental.pallas import tpu as pltpu

_THRESHOLD = 25.0
_BM = 1024


def _body(x_ref, W1_ref, b1_ref, WcT_ref, bc_ref, W2_hbm, b2_ref, WfT_hbm,
          bf_ref, outT_ref, w2_vmem, wfT_vmem, sem2, semf):
    i = pl.program_id(0)

    @pl.when(i == 0)
    def _start_weight_fetch():
        pltpu.make_async_copy(W2_hbm, w2_vmem, sem2).start()
        pltpu.make_async_copy(WfT_hbm, wfT_vmem, semf).start()

    x = x_ref[...]
    h = jnp.maximum(
        jnp.dot(x, W1_ref[...], preferred_element_type=jnp.float32)
        + b1_ref[...], 0.0)
    hT = h.T
    cpT = (jnp.dot(WcT_ref[...], hT, preferred_element_type=jnp.float32)
           + bc_ref[...].T)
    mxT = jnp.max(jnp.exp(cpT), axis=0, keepdims=True)

    @pl.when(i == 0)
    def _wait_weight_fetch():
        pltpu.make_async_copy(W2_hbm, w2_vmem, sem2).wait()
        pltpu.make_async_copy(WfT_hbm, wfT_vmem, semf).wait()

    h2 = jnp.maximum(
        jnp.dot(h, w2_vmem[...], preferred_element_type=jnp.float32)
        + b2_ref[...], 0.0)
    foT = (jnp.dot(wfT_vmem[...], h2.T, preferred_element_type=jnp.float32)
           + bf_ref[...].T)
    outT_ref[...] = jnp.where(mxT > _THRESHOLD, cpT, foT)


def kernel(x, W1, b1, Wc, bc, W2, b2, Wf, bf):
    B, D = x.shape
    NC = Wc.shape[1]

    def _full(shape):
        return pl.BlockSpec(shape, lambda i: (0, 0))

    outT = pl.pallas_call(
        _body,
        grid=(B // _BM,),
        in_specs=[
            pl.BlockSpec((_BM, D), lambda i: (i, 0)),
            _full((D, D)),
            _full((1, D)),
            _full((NC, D)),
            _full((1, NC)),
            pl.BlockSpec(memory_space=pl.ANY),
            _full((1, D)),
            pl.BlockSpec(memory_space=pl.ANY),
            _full((1, NC)),
        ],
        out_specs=pl.BlockSpec((NC, _BM), lambda i: (0, i)),
        out_shape=jax.ShapeDtypeStruct((NC, B), jnp.float32),
        scratch_shapes=[
            pltpu.VMEM((D, D), jnp.float32),
            pltpu.VMEM((NC, D), jnp.float32),
            pltpu.SemaphoreType.DMA,
            pltpu.SemaphoreType.DMA,
        ],
        compiler_params=pltpu.CompilerParams(
            dimension_semantics=("arbitrary",)),
    )(x, W1, b1.reshape(1, D), Wc.T, bc.reshape(1, NC), W2,
      b2.reshape(1, D), Wf.T, bf.reshape(1, NC))
    return outT.T
